# baseline (speedup 1.0000x reference)
import functools

import jax
import jax.numpy as jnp
from jax import lax
from jax.experimental import pallas as pl
from jax.experimental.pallas import tpu as pltpu
from jax.experimental.pallas import tpu_sc as plsc

_K = 1024
_C = 32
_L = 16


def _scores_idx(x_ref, cb_ref, idx_ref):
    xb = x_ref[0].reshape(_C, -1)
    cb = cb_ref[...]
    cbn = jnp.sum(cb * cb, axis=1, keepdims=True)
    s = lax.dot_general(
        cb, xb, (((1,), (0,)), ((), ())),
        preferred_element_type=jnp.float32,
        precision=lax.Precision.HIGHEST)
    scores = cbn - 2.0 * s
    m = jnp.min(scores, axis=0, keepdims=True)
    iota_k = lax.broadcasted_iota(jnp.int32, scores.shape, 0)
    idx_ref[...] = jnp.min(jnp.where(scores == m, iota_k, _K), axis=0)


def _idx_body_cbt(x_ref, cb_ref, idx_ref, cbt_ref):
    _scores_idx(x_ref, cb_ref, idx_ref)

    @pl.when(pl.program_id(0) == 0)
    def _():
        cbt_ref[...] = jnp.reshape(cb_ref[...].T, (_C * _K,))


def _compute_indices(x, codebook, n0, nb, emit_cbt):
    N, C, H, W = x.shape
    hw = H * W
    in_specs = [
        pl.BlockSpec((1, C, H, W), lambda i: (i + n0, 0, 0, 0)),
        pl.BlockSpec((_K, _C), lambda i: (0, 0)),
    ]
    if emit_cbt:
        return pl.pallas_call(
            _idx_body_cbt,
            grid=(nb,),
            in_specs=in_specs,
            out_specs=[
                pl.BlockSpec((hw,), lambda i: (i,)),
                pl.BlockSpec((_C * _K,), lambda i: (0,)),
            ],
            out_shape=[
                jax.ShapeDtypeStruct((nb * hw,), jnp.int32),
                jax.ShapeDtypeStruct((_C * _K,), jnp.float32),
            ],
        )(x, codebook)
    return pl.pallas_call(
        _scores_idx,
        grid=(nb,),
        in_specs=in_specs,
        out_specs=pl.BlockSpec((hw,), lambda i: (i,)),
        out_shape=jax.ShapeDtypeStruct((nb * hw,), jnp.int32),
    )(x, codebook)


def _sc_gather(idx_flat, cbt_flat, nb, hw):
    info = plsc.get_sparse_core_info()
    nc, ns = info.num_cores, info.num_subcores
    nw = nc * ns
    cpw = (nb * _C) // nw
    tpb = _C // cpw
    mesh = plsc.VectorSubcoreMesh(core_axis_name="c", subcore_axis_name="s")

    @functools.partial(
        pl.kernel,
        out_type=jax.ShapeDtypeStruct((nb, _C, hw), jnp.float32),
        mesh=mesh,
        compiler_params=pltpu.CompilerParams(
            needs_layout_passes=False, use_tc_tiling_on_sc=False),
        scratch_types=[
            pltpu.VMEM((hw,), jnp.int32),
            pltpu.VMEM((cpw * _K,), jnp.float32),
            pltpu.VMEM((cpw, hw), jnp.float32),
            pltpu.SemaphoreType.DMA,
        ],
    )
    def gather_kernel(idx_hbm, cbt_hbm, out_hbm, idx_v, cbc_v, buf_v, sem):
        wid = lax.axis_index("s") * nc + lax.axis_index("c")
        n = wid // tpb
        c0 = (wid - n * tpb) * cpw
        copies = [pltpu.async_copy(idx_hbm.at[pl.ds(n * hw, hw)], idx_v, sem),
                  pltpu.async_copy(cbt_hbm.at[pl.ds(c0 * _K, cpw * _K)],
                                   cbc_v, sem)]
        for cp in copies:
            cp.wait()
        for j in range(hw // _L):
            idx16 = idx_v[pl.ds(j * _L, _L)]
            for i in range(cpw):
                buf_v[i, pl.ds(j * _L, _L)] = plsc.load_gather(
                    cbc_v, [idx16 + i * _K])
        pltpu.sync_copy(buf_v, out_hbm.at[n, pl.ds(c0, cpw), :])

    return gather_kernel(idx_flat, cbt_flat)


def kernel(x, codebook):
    N, C, H, W = x.shape
    hw = H * W
    nb = N // 2
    idx_a, cbt = _compute_indices(x, codebook, 0, nb, True)
    idx_b = _compute_indices(x, codebook, nb, N - nb, False)
    y_a = _sc_gather(idx_a, cbt, nb, hw)
    y_b = _sc_gather(idx_b, cbt, N - nb, hw)
    y = jnp.concatenate([y_a, y_b], axis=0).reshape(N, C, H, W)
    return (y, y)


# speedup vs baseline: 1.1707x; 1.1707x over previous
import functools

import jax
import jax.numpy as jnp
from jax import lax
from jax.experimental import pallas as pl
from jax.experimental.pallas import tpu as pltpu
from jax.experimental.pallas import tpu_sc as plsc

_K = 1024
_C = 32
_L = 16


def _idx_body(x_ref, cb_ref, idx_ref, cbt_ref):
    xb = x_ref[0].reshape(_C, -1)
    cb = cb_ref[...]
    cbn = jnp.sum(cb * cb, axis=1, keepdims=True)
    s = lax.dot_general(
        cb, xb, (((1,), (0,)), ((), ())),
        preferred_element_type=jnp.float32,
        precision=lax.Precision.HIGHEST)
    scores = cbn - 2.0 * s
    m = jnp.min(scores, axis=0, keepdims=True)
    iota_k = lax.broadcasted_iota(jnp.int32, scores.shape, 0)
    idx_ref[...] = jnp.min(jnp.where(scores == m, iota_k, _K), axis=0)

    @pl.when(pl.program_id(0) == 0)
    def _():
        cbt_ref[...] = jnp.reshape(cb.T, (_C * _K,))


def _compute_indices(x, codebook):
    N, C, H, W = x.shape
    hw = H * W
    idx, cbt = pl.pallas_call(
        _idx_body,
        grid=(N,),
        in_specs=[
            pl.BlockSpec((1, C, H, W), lambda i: (i, 0, 0, 0)),
            pl.BlockSpec((_K, _C), lambda i: (0, 0)),
        ],
        out_specs=[
            pl.BlockSpec((hw,), lambda i: (i,)),
            pl.BlockSpec((_C * _K,), lambda i: (0,)),
        ],
        out_shape=[
            jax.ShapeDtypeStruct((N * hw,), jnp.int32),
            jax.ShapeDtypeStruct((_C * _K,), jnp.float32),
        ],
    )(x, codebook)
    return idx, cbt


def _sc_gather(idx_flat, cbt_flat, N, hw):
    info = plsc.get_sparse_core_info()
    nc, ns = info.num_cores, info.num_subcores
    nw = nc * ns
    cpw = (N * _C) // nw
    tpb = _C // cpw
    mesh = plsc.VectorSubcoreMesh(core_axis_name="c", subcore_axis_name="s")

    @functools.partial(
        pl.kernel,
        out_type=jax.ShapeDtypeStruct((N, _C, hw), jnp.float32),
        mesh=mesh,
        compiler_params=pltpu.CompilerParams(
            needs_layout_passes=False, use_tc_tiling_on_sc=False),
        scratch_types=[
            pltpu.VMEM((hw,), jnp.int32),
            pltpu.VMEM((cpw * _K,), jnp.float32),
            pltpu.VMEM((cpw, hw), jnp.float32),
            pltpu.SemaphoreType.DMA,
        ],
    )
    def gather_kernel(idx_hbm, cbt_hbm, out_hbm, idx_v, cbc_v, buf_v, sem):
        wid = lax.axis_index("s") * nc + lax.axis_index("c")
        n = wid // tpb
        c0 = (wid - n * tpb) * cpw
        copies = [pltpu.async_copy(idx_hbm.at[pl.ds(n * hw, hw)], idx_v, sem),
                  pltpu.async_copy(cbt_hbm.at[pl.ds(c0 * _K, cpw * _K)],
                                   cbc_v, sem)]
        for cp in copies:
            cp.wait()
        for j in range(hw // _L):
            idx16 = idx_v[pl.ds(j * _L, _L)]
            for i in range(cpw):
                buf_v[i, pl.ds(j * _L, _L)] = plsc.load_gather(
                    cbc_v, [idx16 + i * _K])
        pltpu.sync_copy(buf_v, out_hbm.at[n, pl.ds(c0, cpw), :])

    return gather_kernel(idx_flat, cbt_flat)


def kernel(x, codebook):
    N, C, H, W = x.shape
    hw = H * W
    idx, cbt = _compute_indices(x, codebook)
    y = _sc_gather(idx, cbt, N, hw)
    y = y.reshape(N, C, H, W)
    return (y, y)
